# Initial kernel scaffold; baseline (speedup 1.0000x reference)
import jax
import jax.numpy as jnp
from jax.experimental import pallas as pl


def kernel(inputs, edge_index, W1, b1, W2, b2, W3, b3, Wres, bres, Wop, bop):
    raise NotImplementedError("write your pallas kernel here")


# speedup vs baseline: 4.7967x; 4.7967x over previous
import functools

import jax
import jax.numpy as jnp
from jax import lax
from jax.experimental import pallas as pl
from jax.experimental.pallas import tpu as pltpu
from jax.experimental.pallas import tpu_sc as plsc

N = 10000
E = 320000
D = 128
NCLS = 40

NC = 2
NS = 16
EPC = E // NC
EPT = EPC // NS
CHUNK = 128
NFULL = EPT // CHUNK
TAIL = EPT - NFULL * CHUNK
RPT = 624
REM = N - NS * RPT
_ROW_CHUNKS = [(0, 128), (128, 128), (256, 128), (384, 128), (512, 112)]

_f32 = jnp.float32


def _tile_row_copies(s, copy_chunk, copy_rem):
    r0 = s * RPT
    for off, size in _ROW_CHUNKS:
        copy_chunk(r0 + off, size)

    @pl.when(s == 0)
    def _():
        copy_rem(N - REM, REM)


def _zero_vmem_rows(ref, nrows, width):
    zero16 = jnp.zeros((16,), _f32)

    def zr(i, _):
        for j in range(width // 16):
            ref[i, pl.ds(j * 16, 16)] = zero16
        return 0

    lax.fori_loop(0, nrows, zr, 0)


def _sc_degrees(src, dst):
    mesh = plsc.VectorSubcoreMesh(core_axis_name="c", subcore_axis_name="s")

    @functools.partial(
        pl.kernel,
        mesh=mesh,
        out_type=(
            jax.ShapeDtypeStruct((NC * N,), _f32),
            jax.ShapeDtypeStruct((NC * N,), _f32),
        ),
        scratch_types=[
            pltpu.VMEM((CHUNK,), jnp.int32),
            pltpu.VMEM((CHUNK,), jnp.int32),
            pltpu.VMEM((TAIL,), jnp.int32),
            pltpu.VMEM((TAIL,), jnp.int32),
            pltpu.VMEM((CHUNK,), _f32),
            pltpu.VMEM((CHUNK,), _f32),
            pltpu.VMEM_SHARED((N,), _f32),
            pltpu.VMEM_SHARED((N,), _f32),
        ],
    )
    def deg_kernel(src_hbm, dst_hbm, outs_hbm, outd_hbm,
                   sidx, didx, sidx_t, didx_t, ones, zeros, acc_s, acc_d):
        c = lax.axis_index("c")
        s = lax.axis_index("s")

        one16 = jnp.full((16,), 1.0, _f32)
        zero16 = jnp.zeros((16,), _f32)

        def fill(i, _):
            ones[pl.ds(i * 16, 16)] = one16
            zeros[pl.ds(i * 16, 16)] = zero16
            return 0

        lax.fori_loop(0, CHUNK // 16, fill, 0)

        def zcopy(row, size):
            pltpu.sync_copy(zeros.at[pl.ds(0, size)],
                            acc_s.at[pl.ds(row, size)])
            pltpu.sync_copy(zeros.at[pl.ds(0, size)],
                            acc_d.at[pl.ds(row, size)])

        _tile_row_copies(s, zcopy, zcopy)
        plsc.subcore_barrier()

        e_base = c * EPC + s * EPT

        def step(k, _):
            e0 = e_base + k * CHUNK
            pltpu.sync_copy(src_hbm.at[pl.ds(e0, CHUNK)], sidx)
            pltpu.sync_copy(dst_hbm.at[pl.ds(e0, CHUNK)], didx)
            pltpu.sync_copy(ones, acc_s.at[sidx], add=True)
            pltpu.sync_copy(ones, acc_d.at[didx], add=True)
            return 0

        lax.fori_loop(0, NFULL, step, 0)

        et = e_base + NFULL * CHUNK
        pltpu.sync_copy(src_hbm.at[pl.ds(et, TAIL)], sidx_t)
        pltpu.sync_copy(dst_hbm.at[pl.ds(et, TAIL)], didx_t)
        pltpu.sync_copy(ones.at[pl.ds(0, TAIL)], acc_s.at[sidx_t], add=True)
        pltpu.sync_copy(ones.at[pl.ds(0, TAIL)], acc_d.at[didx_t], add=True)
        plsc.subcore_barrier()

        def wcopy(row, size):
            pltpu.sync_copy(acc_s.at[pl.ds(row, size)], zeros.at[pl.ds(0, size)])
            pltpu.sync_copy(zeros.at[pl.ds(0, size)],
                            outs_hbm.at[pl.ds(c * N + row, size)])
            pltpu.sync_copy(acc_d.at[pl.ds(row, size)], ones.at[pl.ds(0, size)])
            pltpu.sync_copy(ones.at[pl.ds(0, size)],
                            outd_hbm.at[pl.ds(c * N + row, size)])

        _tile_row_copies(s, wcopy, wcopy)

    return deg_kernel(src, dst)


def _sc_aggregate(h, src, dst):
    mesh = plsc.VectorSubcoreMesh(core_axis_name="c", subcore_axis_name="s")

    @functools.partial(
        pl.kernel,
        mesh=mesh,
        out_type=jax.ShapeDtypeStruct((NC, N, D), _f32),
        scratch_types=[
            pltpu.VMEM((CHUNK,), jnp.int32),
            pltpu.VMEM((CHUNK,), jnp.int32),
            pltpu.VMEM((TAIL,), jnp.int32),
            pltpu.VMEM((TAIL,), jnp.int32),
            pltpu.VMEM((CHUNK, D), _f32),
            pltpu.VMEM((TAIL, D), _f32),
            pltpu.VMEM_SHARED((N, D), _f32),
            pltpu.SemaphoreType.DMA,
        ],
    )
    def agg_kernel(h_hbm, src_hbm, dst_hbm, out_hbm,
                   sidx, didx, sidx_t, didx_t, rows, rows_t, acc, sem):
        c = lax.axis_index("c")
        s = lax.axis_index("s")

        _zero_vmem_rows(rows, CHUNK, D)

        def zcopy(row, size):
            pltpu.sync_copy(rows.at[pl.ds(0, size)],
                            acc.at[pl.ds(row, size)])

        _tile_row_copies(s, zcopy, zcopy)
        plsc.subcore_barrier()

        e_base = c * EPC + s * EPT

        def step(k, _):
            e0 = e_base + k * CHUNK
            pltpu.sync_copy(src_hbm.at[pl.ds(e0, CHUNK)], sidx)
            pltpu.sync_copy(dst_hbm.at[pl.ds(e0, CHUNK)], didx)
            pltpu.async_copy(h_hbm.at[sidx], rows, sem).wait()
            pltpu.sync_copy(rows, acc.at[didx], add=True)
            return 0

        lax.fori_loop(0, NFULL, step, 0)

        et = e_base + NFULL * CHUNK
        pltpu.sync_copy(src_hbm.at[pl.ds(et, TAIL)], sidx_t)
        pltpu.sync_copy(dst_hbm.at[pl.ds(et, TAIL)], didx_t)
        pltpu.async_copy(h_hbm.at[sidx_t], rows_t, sem).wait()
        pltpu.sync_copy(rows_t, acc.at[didx_t], add=True)
        plsc.subcore_barrier()

        def wcopy(row, size):
            pltpu.sync_copy(acc.at[pl.ds(row, size)],
                            out_hbm.at[c, pl.ds(row, size)])

        _tile_row_copies(s, wcopy, wcopy)

    return agg_kernel(h, src, dst)


_BLK = 1000


def _row_spec(width):
    return pl.BlockSpec((_BLK, width), lambda i: (i, 0))


def _full_spec(r, ccols):
    return pl.BlockSpec((r, ccols), lambda i: (0, 0))


def _tc_prepare(x, ds0, ds1, dd0, dd1, wres, bres):
    def body(x_ref, a0, a1, b0, b1, w_ref, bias_ref,
             h_ref, res_ref, rss_ref, rsd_ref):
        rs = lax.rsqrt(jnp.maximum(a0[...] + a1[...], 1.0))
        rd = lax.rsqrt(jnp.maximum(b0[...] + b1[...], 1.0))
        rss_ref[...] = rs
        rsd_ref[...] = rd
        x_ = x_ref[...]
        h_ref[...] = x_ * rs
        res_ref[...] = (
            jnp.dot(x_, w_ref[...], preferred_element_type=_f32)
            + bias_ref[...]
        )

    return pl.pallas_call(
        body,
        grid=(N // _BLK,),
        in_specs=[
            _row_spec(D),
            _row_spec(1), _row_spec(1), _row_spec(1), _row_spec(1),
            _full_spec(D, D), _full_spec(1, D),
        ],
        out_specs=[_row_spec(D), _row_spec(D), _row_spec(1), _row_spec(1)],
        out_shape=[
            jax.ShapeDtypeStruct((N, D), _f32),
            jax.ShapeDtypeStruct((N, D), _f32),
            jax.ShapeDtypeStruct((N, 1), _f32),
            jax.ShapeDtypeStruct((N, 1), _f32),
        ],
    )(x, ds0, ds1, dd0, dd1, wres, bres)


def _tc_layer(p0, p1, rsd, rss, w, b):
    def body(a_ref, b_ref, rd_ref, rs_ref, w_ref, bias_ref, o_ref):
        z = (a_ref[...] + b_ref[...]) * rd_ref[...]
        z = jnp.dot(z, w_ref[...], preferred_element_type=_f32) + bias_ref[...]
        o_ref[...] = jnp.maximum(z, 0.0) * rs_ref[...]

    return pl.pallas_call(
        body,
        grid=(N // _BLK,),
        in_specs=[
            _row_spec(D), _row_spec(D), _row_spec(1), _row_spec(1),
            _full_spec(D, D), _full_spec(1, D),
        ],
        out_specs=_row_spec(D),
        out_shape=jax.ShapeDtypeStruct((N, D), _f32),
    )(p0, p1, rsd, rss, w, b)


def _tc_final(p0, p1, rsd, res, w3, b3, wop, bop):
    def body(a_ref, b_ref, rd_ref, res_ref, w3_ref, b3_ref,
             wop_ref, bop_ref, o_ref):
        z = (a_ref[...] + b_ref[...]) * rd_ref[...]
        z = jnp.dot(z, w3_ref[...], preferred_element_type=_f32) + b3_ref[...]
        h = jnp.maximum(z + res_ref[...], 0.0)
        o_ref[...] = (
            jnp.dot(h, wop_ref[...], preferred_element_type=_f32)
            + bop_ref[...]
        )

    return pl.pallas_call(
        body,
        grid=(N // _BLK,),
        in_specs=[
            _row_spec(D), _row_spec(D), _row_spec(1), _row_spec(D),
            _full_spec(D, D), _full_spec(1, D),
            _full_spec(D, NCLS), _full_spec(1, NCLS),
        ],
        out_specs=_row_spec(NCLS),
        out_shape=jax.ShapeDtypeStruct((N, NCLS), _f32),
    )(p0, p1, rsd, res, w3, b3, wop, bop)


def kernel(inputs, edge_index, W1, b1, W2, b2, W3, b3, Wres, bres, Wop, bop):
    src = edge_index[0].astype(jnp.int32)
    dst = edge_index[1].astype(jnp.int32)

    degs, degd = _sc_degrees(src, dst)
    ds0 = degs[:N].reshape(N, 1)
    ds1 = degs[N:].reshape(N, 1)
    dd0 = degd[:N].reshape(N, 1)
    dd1 = degd[N:].reshape(N, 1)

    h0, res, rss, rsd = _tc_prepare(
        inputs, ds0, ds1, dd0, dd1, Wres, bres.reshape(1, D))

    p = _sc_aggregate(h0, src, dst)
    h1 = _tc_layer(p[0], p[1], rsd, rss, W1, b1.reshape(1, D))

    p = _sc_aggregate(h1, src, dst)
    h2 = _tc_layer(p[0], p[1], rsd, rss, W2, b2.reshape(1, D))

    p = _sc_aggregate(h2, src, dst)
    out = _tc_final(p[0], p[1], rsd, res, W3, b3.reshape(1, D),
                    Wop, bop.reshape(1, NCLS))
    return out


# speedup vs baseline: 10.2318x; 2.1331x over previous
import functools

import jax
import jax.numpy as jnp
from jax import lax
from jax.experimental import pallas as pl
from jax.experimental.pallas import tpu as pltpu
from jax.experimental.pallas import tpu_sc as plsc

N = 10000
E = 320000
D = 128
NCLS = 40

NC = 2
NS = 16
NW = NC * NS
CHUNK = 80
RPW = 128
EPAD = NW * RPW * CHUNK
NPAD = 10112
RPT = NPAD // NS
NSLOT = 4
GRP = 8
NGRP = RPW // GRP
_ROW_CHUNKS = [(i * CHUNK, CHUNK) for i in range(RPT // CHUNK)]
if RPT % CHUNK:
    _ROW_CHUNKS.append(((RPT // CHUNK) * CHUNK, RPT % CHUNK))

_f32 = jnp.float32


def _sc_degrees(src3d, dst3d):
    mesh = plsc.VectorSubcoreMesh(core_axis_name="c", subcore_axis_name="s")

    @functools.partial(
        pl.kernel,
        mesh=mesh,
        out_type=(
            jax.ShapeDtypeStruct((NC * NPAD,), _f32),
            jax.ShapeDtypeStruct((NC * NPAD,), _f32),
        ),
        scratch_types=[
            pltpu.VMEM((RPW, CHUNK), jnp.int32),
            pltpu.VMEM((RPW, CHUNK), jnp.int32),
            pltpu.VMEM((CHUNK,), _f32),
            pltpu.VMEM((CHUNK,), _f32),
            pltpu.VMEM_SHARED((NPAD,), _f32),
            pltpu.VMEM_SHARED((NPAD,), _f32),
            pltpu.SemaphoreType.DMA,
            pltpu.SemaphoreType.DMA,
        ],
    )
    def deg_kernel(src_hbm, dst_hbm, outs_hbm, outd_hbm,
                   sidx, didx, ones, zeros, acc_s, acc_d, sem_s, sem_d):
        c = lax.axis_index("c")
        s = lax.axis_index("s")
        w = c * NS + s

        pltpu.sync_copy(src_hbm.at[w], sidx)
        pltpu.sync_copy(dst_hbm.at[w], didx)
        one16 = jnp.full((16,), 1.0, _f32)
        zero16 = jnp.zeros((16,), _f32)

        def fill(i, _):
            ones[pl.ds(i * 16, 16)] = one16
            zeros[pl.ds(i * 16, 16)] = zero16
            return 0

        lax.fori_loop(0, CHUNK // 16, fill, 0)

        r0 = s * RPT
        for off, size in _ROW_CHUNKS:
            pltpu.sync_copy(zeros.at[pl.ds(0, size)],
                            acc_s.at[pl.ds(r0 + off, size)])
            pltpu.sync_copy(zeros.at[pl.ds(0, size)],
                            acc_d.at[pl.ds(r0 + off, size)])
        plsc.subcore_barrier()

        GB = 8

        def group(g, _):
            for b in range(GB):
                r = g * GB + b
                pltpu.async_copy(ones, acc_s.at[sidx.at[r]], sem_s, add=True)
                pltpu.async_copy(ones, acc_d.at[didx.at[r]], sem_d, add=True)
            for b in range(GB):
                r = g * GB + b
                pltpu.make_async_copy(ones, acc_s.at[sidx.at[r]], sem_s).wait()
                pltpu.make_async_copy(ones, acc_d.at[didx.at[r]], sem_d).wait()
            return 0

        lax.fori_loop(0, RPW // GB, group, 0)
        plsc.subcore_barrier()

        for off, size in _ROW_CHUNKS:
            row = r0 + off
            pltpu.sync_copy(acc_s.at[pl.ds(row, size)], zeros.at[pl.ds(0, size)])
            pltpu.sync_copy(zeros.at[pl.ds(0, size)],
                            outs_hbm.at[pl.ds(c * NPAD + row, size)])
            pltpu.sync_copy(acc_d.at[pl.ds(row, size)], ones.at[pl.ds(0, size)])
            pltpu.sync_copy(ones.at[pl.ds(0, size)],
                            outd_hbm.at[pl.ds(c * NPAD + row, size)])

    return deg_kernel(src3d, dst3d)


def _sc_aggregate(h, src3d, dst3d):
    mesh = plsc.VectorSubcoreMesh(core_axis_name="c", subcore_axis_name="s")

    @functools.partial(
        pl.kernel,
        mesh=mesh,
        out_type=jax.ShapeDtypeStruct((NC, NPAD, D), _f32),
        scratch_types=[
            pltpu.VMEM((GRP, CHUNK), jnp.int32),
            pltpu.VMEM((GRP, CHUNK), jnp.int32),
            pltpu.VMEM((GRP, CHUNK), jnp.int32),
            pltpu.VMEM((GRP, CHUNK), jnp.int32),
            pltpu.VMEM((CHUNK, D), _f32),
            pltpu.VMEM((CHUNK, D), _f32),
            pltpu.VMEM((CHUNK, D), _f32),
            pltpu.VMEM((CHUNK, D), _f32),
            pltpu.VMEM_SHARED((NPAD, D), _f32),
            pltpu.SemaphoreType.DMA,
            pltpu.SemaphoreType.DMA,
            pltpu.SemaphoreType.DMA,
            pltpu.SemaphoreType.DMA,
            pltpu.SemaphoreType.DMA,
            pltpu.SemaphoreType.DMA,
        ],
    )
    def agg_kernel(h_hbm, src_hbm, dst_hbm, out_hbm,
                   sb0, sb1, db0, db1, r0buf, r1buf, r2buf, r3buf, acc,
                   sem0, sem1, sem2, sem3, isem0, isem1):
        c = lax.axis_index("c")
        s = lax.axis_index("s")
        w = c * NS + s
        rows = [r0buf, r1buf, r2buf, r3buf]
        sems = [sem0, sem1, sem2, sem3]
        sbuf = [sb0, sb1]
        dbuf = [db0, db1]
        isem = [isem0, isem1]

        def fire_idx(grp, par, sync=False):
            if sync:
                pltpu.sync_copy(src_hbm.at[w, pl.ds(grp * GRP, GRP)], sbuf[par])
                pltpu.sync_copy(dst_hbm.at[w, pl.ds(grp * GRP, GRP)], dbuf[par])
            else:
                pltpu.async_copy(src_hbm.at[w, pl.ds(grp * GRP, GRP)],
                                 sbuf[par], isem[par])
                pltpu.async_copy(dst_hbm.at[w, pl.ds(grp * GRP, GRP)],
                                 dbuf[par], isem[par])

        def wait_idx(par):
            pltpu.make_async_copy(src_hbm.at[w, pl.ds(0, GRP)],
                                  sbuf[par], isem[par]).wait()
            pltpu.make_async_copy(dst_hbm.at[w, pl.ds(0, GRP)],
                                  dbuf[par], isem[par]).wait()

        def fire_gather(sidx_row, b):
            pltpu.async_copy(h_hbm.at[sidx_row], rows[b], sems[b])

        def wait_gather(b):
            pltpu.make_async_copy(h_hbm.at[pl.ds(0, CHUNK)],
                                  rows[b], sems[b]).wait()

        zero16 = jnp.zeros((16,), _f32)

        def zr(i, _):
            for j in range(D // 16):
                r0buf[i, pl.ds(j * 16, 16)] = zero16
            return 0

        lax.fori_loop(0, CHUNK, zr, 0)
        r0 = s * RPT
        for off, size in _ROW_CHUNKS:
            pltpu.sync_copy(r0buf.at[pl.ds(0, size)],
                            acc.at[pl.ds(r0 + off, size)])
        plsc.subcore_barrier()

        fire_idx(0, 0, sync=True)
        fire_idx(1, 1)
        for b in range(NSLOT):
            fire_gather(sb0.at[b], b)

        def run_group(g, par):
            nxt = 1 - par
            for j in range(GRP):
                if j == NSLOT:
                    @pl.when(g < NGRP - 1)
                    def _():
                        wait_idx(nxt)
                b = j % NSLOT
                wait_gather(b)
                pltpu.sync_copy(rows[b], acc.at[dbuf[par].at[j]], add=True)
                if j < NSLOT:
                    fire_gather(sbuf[par].at[j + NSLOT], b)
                else:
                    @pl.when(g < NGRP - 1)
                    def _():
                        fire_gather(sbuf[nxt].at[j - NSLOT], b)

            @pl.when(g < NGRP - 2)
            def _():
                fire_idx(g + 2, par)

        def pair(gg, _):
            run_group(2 * gg, 0)
            run_group(2 * gg + 1, 1)
            return 0

        lax.fori_loop(0, NGRP // 2, pair, 0)
        plsc.subcore_barrier()

        for off, size in _ROW_CHUNKS:
            row = r0 + off
            pltpu.sync_copy(acc.at[pl.ds(row, size)],
                            out_hbm.at[c, pl.ds(row, size)])

    return agg_kernel(h, src3d, dst3d)


_BLK = 632


def _row_spec(width):
    return pl.BlockSpec((_BLK, width), lambda i: (i, 0))


def _full_spec(r, ccols):
    return pl.BlockSpec((r, ccols), lambda i: (0, 0))


def _tc_prepare(x, ds0, ds1, dd0, dd1, wres, bres):
    def body(x_ref, a0, a1, b0, b1, w_ref, bias_ref,
             h_ref, res_ref, rss_ref, rsd_ref):
        rs = lax.rsqrt(jnp.maximum(a0[...] + a1[...], 1.0))
        rd = lax.rsqrt(jnp.maximum(b0[...] + b1[...], 1.0))
        rss_ref[...] = rs
        rsd_ref[...] = rd
        x_ = x_ref[...]
        h_ref[...] = x_ * rs
        res_ref[...] = (
            jnp.dot(x_, w_ref[...], preferred_element_type=_f32)
            + bias_ref[...]
        )

    return pl.pallas_call(
        body,
        grid=(NPAD // _BLK,),
        in_specs=[
            _row_spec(D),
            _row_spec(1), _row_spec(1), _row_spec(1), _row_spec(1),
            _full_spec(D, D), _full_spec(1, D),
        ],
        out_specs=[_row_spec(D), _row_spec(D), _row_spec(1), _row_spec(1)],
        out_shape=[
            jax.ShapeDtypeStruct((NPAD, D), _f32),
            jax.ShapeDtypeStruct((NPAD, D), _f32),
            jax.ShapeDtypeStruct((NPAD, 1), _f32),
            jax.ShapeDtypeStruct((NPAD, 1), _f32),
        ],
    )(x, ds0, ds1, dd0, dd1, wres, bres)


def _tc_layer(p0, p1, rsd, rss, w, b):
    def body(a_ref, b_ref, rd_ref, rs_ref, w_ref, bias_ref, o_ref):
        z = (a_ref[...] + b_ref[...]) * rd_ref[...]
        z = jnp.dot(z, w_ref[...], preferred_element_type=_f32) + bias_ref[...]
        o_ref[...] = jnp.maximum(z, 0.0) * rs_ref[...]

    return pl.pallas_call(
        body,
        grid=(NPAD // _BLK,),
        in_specs=[
            _row_spec(D), _row_spec(D), _row_spec(1), _row_spec(1),
            _full_spec(D, D), _full_spec(1, D),
        ],
        out_specs=_row_spec(D),
        out_shape=jax.ShapeDtypeStruct((NPAD, D), _f32),
    )(p0, p1, rsd, rss, w, b)


def _tc_final(p0, p1, rsd, res, w3, b3, wop, bop):
    def body(a_ref, b_ref, rd_ref, res_ref, w3_ref, b3_ref,
             wop_ref, bop_ref, o_ref):
        z = (a_ref[...] + b_ref[...]) * rd_ref[...]
        z = jnp.dot(z, w3_ref[...], preferred_element_type=_f32) + b3_ref[...]
        h = jnp.maximum(z + res_ref[...], 0.0)
        o_ref[...] = (
            jnp.dot(h, wop_ref[...], preferred_element_type=_f32)
            + bop_ref[...]
        )

    return pl.pallas_call(
        body,
        grid=(NPAD // _BLK,),
        in_specs=[
            _row_spec(D), _row_spec(D), _row_spec(1), _row_spec(D),
            _full_spec(D, D), _full_spec(1, D),
            _full_spec(D, NCLS), _full_spec(1, NCLS),
        ],
        out_specs=_row_spec(NCLS),
        out_shape=jax.ShapeDtypeStruct((NPAD, NCLS), _f32),
    )(p0, p1, rsd, res, w3, b3, wop, bop)


def kernel(inputs, edge_index, W1, b1, W2, b2, W3, b3, Wres, bres, Wop, bop):
    src = edge_index[0].astype(jnp.int32)
    dst = edge_index[1].astype(jnp.int32)

    pad_idx = N + (jnp.arange(EPAD - E, dtype=jnp.int32) % (NPAD - N))
    src3d = jnp.concatenate([src, pad_idx]).reshape(NW, RPW, CHUNK)
    dst3d = jnp.concatenate([dst, pad_idx]).reshape(NW, RPW, CHUNK)
    x = jnp.zeros((NPAD, D), _f32).at[:N].set(inputs)

    degs, degd = _sc_degrees(src3d, dst3d)
    ds0 = degs[:NPAD].reshape(NPAD, 1)
    ds1 = degs[NPAD:].reshape(NPAD, 1)
    dd0 = degd[:NPAD].reshape(NPAD, 1)
    dd1 = degd[NPAD:].reshape(NPAD, 1)

    h0, res, rss, rsd = _tc_prepare(
        x, ds0, ds1, dd0, dd1, Wres, bres.reshape(1, D))

    p = _sc_aggregate(h0, src3d, dst3d)
    h1 = _tc_layer(p[0], p[1], rsd, rss, W1, b1.reshape(1, D))

    p = _sc_aggregate(h1, src3d, dst3d)
    h2 = _tc_layer(p[0], p[1], rsd, rss, W2, b2.reshape(1, D))

    p = _sc_aggregate(h2, src3d, dst3d)
    out = _tc_final(p[0], p[1], rsd, res, W3, b3.reshape(1, D),
                    Wop, bop.reshape(1, NCLS))
    return out[:N]


# speedup vs baseline: 10.2581x; 1.0026x over previous
import functools

import jax
import jax.numpy as jnp
from jax import lax
from jax.experimental import pallas as pl
from jax.experimental.pallas import tpu as pltpu
from jax.experimental.pallas import tpu_sc as plsc

N = 10000
E = 320000
D = 128
NCLS = 40

NC = 2
NS = 16
NW = NC * NS
CHUNK = 80
RPW = 128
EPAD = NW * RPW * CHUNK
NPAD = 10112
RPT = NPAD // NS
NSLOT = 4
GRP = 8
NGRP = RPW // GRP
_ROW_CHUNKS = [(i * CHUNK, CHUNK) for i in range(RPT // CHUNK)]
if RPT % CHUNK:
    _ROW_CHUNKS.append(((RPT // CHUNK) * CHUNK, RPT % CHUNK))

_f32 = jnp.float32


def _sc_degrees(src3d, dst3d):
    mesh = plsc.VectorSubcoreMesh(core_axis_name="c", subcore_axis_name="s")

    @functools.partial(
        pl.kernel,
        mesh=mesh,
        out_type=(
            jax.ShapeDtypeStruct((NC * NPAD,), _f32),
            jax.ShapeDtypeStruct((NC * NPAD,), _f32),
        ),
        scratch_types=[
            pltpu.VMEM((RPW, CHUNK), jnp.int32),
            pltpu.VMEM((RPW, CHUNK), jnp.int32),
            pltpu.VMEM((CHUNK,), _f32),
            pltpu.VMEM((CHUNK,), _f32),
            pltpu.VMEM_SHARED((NPAD,), _f32),
            pltpu.VMEM_SHARED((NPAD,), _f32),
            pltpu.SemaphoreType.DMA,
            pltpu.SemaphoreType.DMA,
        ],
    )
    def deg_kernel(src_hbm, dst_hbm, outs_hbm, outd_hbm,
                   sidx, didx, ones, zeros, acc_s, acc_d, sem_s, sem_d):
        c = lax.axis_index("c")
        s = lax.axis_index("s")
        w = c * NS + s

        pltpu.sync_copy(src_hbm.at[w], sidx)
        pltpu.sync_copy(dst_hbm.at[w], didx)
        one16 = jnp.full((16,), 1.0, _f32)
        zero16 = jnp.zeros((16,), _f32)

        def fill(i, _):
            ones[pl.ds(i * 16, 16)] = one16
            zeros[pl.ds(i * 16, 16)] = zero16
            return 0

        lax.fori_loop(0, CHUNK // 16, fill, 0)

        r0 = s * RPT
        for off, size in _ROW_CHUNKS:
            pltpu.sync_copy(zeros.at[pl.ds(0, size)],
                            acc_s.at[pl.ds(r0 + off, size)])
            pltpu.sync_copy(zeros.at[pl.ds(0, size)],
                            acc_d.at[pl.ds(r0 + off, size)])
        plsc.subcore_barrier()

        GB = 8

        def group(g, _):
            for b in range(GB):
                r = g * GB + b
                pltpu.async_copy(ones, acc_s.at[sidx.at[r]], sem_s, add=True)
                pltpu.async_copy(ones, acc_d.at[didx.at[r]], sem_d, add=True)
            for b in range(GB):
                r = g * GB + b
                pltpu.make_async_copy(ones, acc_s.at[sidx.at[r]], sem_s).wait()
                pltpu.make_async_copy(ones, acc_d.at[didx.at[r]], sem_d).wait()
            return 0

        lax.fori_loop(0, RPW // GB, group, 0)
        plsc.subcore_barrier()

        for off, size in _ROW_CHUNKS:
            row = r0 + off
            pltpu.sync_copy(acc_s.at[pl.ds(row, size)], zeros.at[pl.ds(0, size)])
            pltpu.sync_copy(zeros.at[pl.ds(0, size)],
                            outs_hbm.at[pl.ds(c * NPAD + row, size)])
            pltpu.sync_copy(acc_d.at[pl.ds(row, size)], ones.at[pl.ds(0, size)])
            pltpu.sync_copy(ones.at[pl.ds(0, size)],
                            outd_hbm.at[pl.ds(c * NPAD + row, size)])

    return deg_kernel(src3d, dst3d)


def _sc_aggregate(h, src3d, dst3d):
    mesh = plsc.VectorSubcoreMesh(core_axis_name="c", subcore_axis_name="s")

    @functools.partial(
        pl.kernel,
        mesh=mesh,
        out_type=jax.ShapeDtypeStruct((NC, NPAD, D), _f32),
        scratch_types=[
            pltpu.VMEM((GRP, CHUNK), jnp.int32),
            pltpu.VMEM((GRP, CHUNK), jnp.int32),
            pltpu.VMEM((GRP, CHUNK), jnp.int32),
            pltpu.VMEM((GRP, CHUNK), jnp.int32),
            pltpu.VMEM((CHUNK, D), _f32),
            pltpu.VMEM((CHUNK, D), _f32),
            pltpu.VMEM((CHUNK, D), _f32),
            pltpu.VMEM((CHUNK, D), _f32),
            pltpu.VMEM_SHARED((NPAD, D), _f32),
            pltpu.SemaphoreType.DMA,
            pltpu.SemaphoreType.DMA,
            pltpu.SemaphoreType.DMA,
            pltpu.SemaphoreType.DMA,
            pltpu.SemaphoreType.DMA,
            pltpu.SemaphoreType.DMA,
        ],
    )
    def agg_kernel(h_hbm, src_hbm, dst_hbm, out_hbm,
                   sb0, sb1, db0, db1, r0buf, r1buf, r2buf, r3buf, acc,
                   sem0, sem1, sem2, sem3, isem0, isem1):
        c = lax.axis_index("c")
        s = lax.axis_index("s")
        w = c * NS + s
        rows = [r0buf, r1buf, r2buf, r3buf]
        sems = [sem0, sem1, sem2, sem3]
        sbuf = [sb0, sb1]
        dbuf = [db0, db1]
        isem = [isem0, isem1]

        def fire_idx(grp, par, sync=False):
            if sync:
                pltpu.sync_copy(src_hbm.at[w, pl.ds(grp * GRP, GRP)], sbuf[par])
                pltpu.sync_copy(dst_hbm.at[w, pl.ds(grp * GRP, GRP)], dbuf[par])
            else:
                pltpu.async_copy(src_hbm.at[w, pl.ds(grp * GRP, GRP)],
                                 sbuf[par], isem[par])
                pltpu.async_copy(dst_hbm.at[w, pl.ds(grp * GRP, GRP)],
                                 dbuf[par], isem[par])

        def wait_idx(par):
            pltpu.make_async_copy(src_hbm.at[w, pl.ds(0, GRP)],
                                  sbuf[par], isem[par]).wait()
            pltpu.make_async_copy(dst_hbm.at[w, pl.ds(0, GRP)],
                                  dbuf[par], isem[par]).wait()

        def fire_gather(sidx_row, b):
            pltpu.async_copy(h_hbm.at[sidx_row], rows[b], sems[b])

        def wait_gather(b):
            pltpu.make_async_copy(h_hbm.at[pl.ds(0, CHUNK)],
                                  rows[b], sems[b]).wait()

        zero16 = jnp.zeros((16,), _f32)

        def zr(i, _):
            for j in range(D // 16):
                r0buf[i, pl.ds(j * 16, 16)] = zero16
            return 0

        lax.fori_loop(0, CHUNK, zr, 0)
        r0 = s * RPT
        for off, size in _ROW_CHUNKS:
            pltpu.sync_copy(r0buf.at[pl.ds(0, size)],
                            acc.at[pl.ds(r0 + off, size)])
        plsc.subcore_barrier()

        fire_idx(0, 0, sync=True)
        fire_idx(1, 1)
        for b in range(NSLOT):
            fire_gather(sb0.at[b], b)

        def run_group(g, par):
            nxt = 1 - par
            for j in range(GRP):
                if j == NSLOT:
                    @pl.when(g < NGRP - 1)
                    def _():
                        wait_idx(nxt)
                b = j % NSLOT
                wait_gather(b)
                pltpu.sync_copy(rows[b], acc.at[dbuf[par].at[j]], add=True)
                if j < NSLOT:
                    fire_gather(sbuf[par].at[j + NSLOT], b)
                else:
                    @pl.when(g < NGRP - 1)
                    def _():
                        fire_gather(sbuf[nxt].at[j - NSLOT], b)

            @pl.when(g < NGRP - 2)
            def _():
                fire_idx(g + 2, par)

        def pair(gg, _):
            run_group(2 * gg, 0)
            run_group(2 * gg + 1, 1)
            return 0

        lax.fori_loop(0, NGRP // 2, pair, 0)
        plsc.subcore_barrier()

        for off, size in _ROW_CHUNKS:
            row = r0 + off
            pltpu.sync_copy(acc.at[pl.ds(row, size)],
                            out_hbm.at[c, pl.ds(row, size)])

    return agg_kernel(h, src3d, dst3d)


_BLK = 632


def _row_spec(width):
    return pl.BlockSpec((_BLK, width), lambda i: (i, 0))


def _full_spec(r, ccols):
    return pl.BlockSpec((r, ccols), lambda i: (0, 0))


def _tc_prepare(x, ds0, ds1, dd0, dd1):
    def body(x_ref, a0, a1, b0, b1, h_ref, rss_ref, rsd_ref):
        rs = lax.rsqrt(jnp.maximum(a0[...] + a1[...], 1.0))
        rd = lax.rsqrt(jnp.maximum(b0[...] + b1[...], 1.0))
        rss_ref[...] = rs
        rsd_ref[...] = rd
        h_ref[...] = x_ref[...] * rs

    return pl.pallas_call(
        body,
        grid=(NPAD // _BLK,),
        in_specs=[
            _row_spec(D),
            _row_spec(1), _row_spec(1), _row_spec(1), _row_spec(1),
        ],
        out_specs=[_row_spec(D), _row_spec(1), _row_spec(1)],
        out_shape=[
            jax.ShapeDtypeStruct((NPAD, D), _f32),
            jax.ShapeDtypeStruct((NPAD, 1), _f32),
            jax.ShapeDtypeStruct((NPAD, 1), _f32),
        ],
    )(x, ds0, ds1, dd0, dd1)


def _tc_layer(p0, p1, rsd, rss, w, b):
    def body(a_ref, b_ref, rd_ref, rs_ref, w_ref, bias_ref, o_ref):
        z = (a_ref[...] + b_ref[...]) * rd_ref[...]
        z = jnp.dot(z, w_ref[...], preferred_element_type=_f32) + bias_ref[...]
        o_ref[...] = jnp.maximum(z, 0.0) * rs_ref[...]

    return pl.pallas_call(
        body,
        grid=(NPAD // _BLK,),
        in_specs=[
            _row_spec(D), _row_spec(D), _row_spec(1), _row_spec(1),
            _full_spec(D, D), _full_spec(1, D),
        ],
        out_specs=_row_spec(D),
        out_shape=jax.ShapeDtypeStruct((NPAD, D), _f32),
    )(p0, p1, rsd, rss, w, b)


def _tc_final(p0, p1, rsd, x, wres, bres, w3, b3, wop, bop):
    def body(a_ref, b_ref, rd_ref, x_ref, wr_ref, br_ref, w3_ref, b3_ref,
             wop_ref, bop_ref, o_ref):
        z = (a_ref[...] + b_ref[...]) * rd_ref[...]
        z = jnp.dot(z, w3_ref[...], preferred_element_type=_f32) + b3_ref[...]
        res = (jnp.dot(x_ref[...], wr_ref[...], preferred_element_type=_f32)
               + br_ref[...])
        h = jnp.maximum(z + res, 0.0)
        o_ref[...] = (
            jnp.dot(h, wop_ref[...], preferred_element_type=_f32)
            + bop_ref[...]
        )

    return pl.pallas_call(
        body,
        grid=(NPAD // _BLK,),
        in_specs=[
            _row_spec(D), _row_spec(D), _row_spec(1), _row_spec(D),
            _full_spec(D, D), _full_spec(1, D),
            _full_spec(D, D), _full_spec(1, D),
            _full_spec(D, NCLS), _full_spec(1, NCLS),
        ],
        out_specs=_row_spec(NCLS),
        out_shape=jax.ShapeDtypeStruct((NPAD, NCLS), _f32),
    )(p0, p1, rsd, x, wres, bres, w3, b3, wop, bop)


def kernel(inputs, edge_index, W1, b1, W2, b2, W3, b3, Wres, bres, Wop, bop):
    src = edge_index[0].astype(jnp.int32)
    dst = edge_index[1].astype(jnp.int32)

    pad_idx = N + (jnp.arange(EPAD - E, dtype=jnp.int32) % (NPAD - N))
    src3d = jnp.concatenate([src, pad_idx]).reshape(NW, RPW, CHUNK)
    dst3d = jnp.concatenate([dst, pad_idx]).reshape(NW, RPW, CHUNK)
    x = jnp.zeros((NPAD, D), _f32).at[:N].set(inputs)

    degs, degd = _sc_degrees(src3d, dst3d)
    ds0 = degs[:NPAD].reshape(NPAD, 1)
    ds1 = degs[NPAD:].reshape(NPAD, 1)
    dd0 = degd[:NPAD].reshape(NPAD, 1)
    dd1 = degd[NPAD:].reshape(NPAD, 1)

    h0, rss, rsd = _tc_prepare(x, ds0, ds1, dd0, dd1)

    p = _sc_aggregate(h0, src3d, dst3d)
    h1 = _tc_layer(p[0], p[1], rsd, rss, W1, b1.reshape(1, D))

    p = _sc_aggregate(h1, src3d, dst3d)
    h2 = _tc_layer(p[0], p[1], rsd, rss, W2, b2.reshape(1, D))

    p = _sc_aggregate(h2, src3d, dst3d)
    out = _tc_final(p[0], p[1], rsd, x, Wres, bres.reshape(1, D),
                    W3, b3.reshape(1, D), Wop, bop.reshape(1, NCLS))
    return out[:N]
